# Initial kernel scaffold; baseline (speedup 1.0000x reference)
import jax
import jax.numpy as jnp
from jax.experimental import pallas as pl


def kernel(edge_index, adj_values, feats, W, b, ln_gamma, ln_beta):
    raise NotImplementedError("write your pallas kernel here")


# speedup vs baseline: 5.2852x; 5.2852x over previous
import functools

import jax
import jax.numpy as jnp
from jax import lax
from jax.experimental import pallas as pl
from jax.experimental.pallas import tpu as pltpu
from jax.experimental.pallas import tpu_sc as plsc

_N = 10000
_D = 128
_NC = 2
_NS = 16
_NW = _NC * _NS
_C = 128
_ROWS_PER_TILE = 624
_TAIL_BASE = _ROWS_PER_TILE * _NS
_TAIL = _N - _TAIL_BASE


def _linear_kernel(f_ref, w_ref, b_ref, o_ref):
    o_ref[...] = lax.dot_general(
        f_ref[...], w_ref[...],
        dimension_numbers=(((1,), (1,)), ((), ())),
        preferred_element_type=jnp.float32,
        precision=lax.Precision.HIGHEST,
    ) + b_ref[...]


def _ln_kernel(p_ref, g_ref, be_ref, o_ref):
    h = p_ref[0] + p_ref[1]
    m = jnp.mean(h, axis=-1, keepdims=True)
    c = h - m
    v = jnp.mean(c * c, axis=-1, keepdims=True)
    y = c * lax.rsqrt(v + 1e-5) * g_ref[...] + be_ref[...]
    o_ref[...] = jnp.maximum(y, 0.0)


def _make_agg(E):
    nchunks = E // _C
    base_chunks = nchunks // _NW
    extra = nchunks % _NW
    mesh = plsc.VectorSubcoreMesh(core_axis_name="c", subcore_axis_name="s")

    @functools.partial(
        pl.kernel, mesh=mesh,
        out_type=jax.ShapeDtypeStruct((_NC, _N, _D), jnp.float32),
        scratch_types=[
            pltpu.VMEM((_C,), jnp.int32),
            pltpu.VMEM((_C,), jnp.int32),
            pltpu.VMEM((_C,), jnp.float32),
            pltpu.VMEM((_C, _D), jnp.float32),
            pltpu.VMEM_SHARED((_N, _D), jnp.float32),
            pltpu.SemaphoreType.DMA,
        ],
    )
    def agg(support, src_h, dst_h, adj_h, zeros_h, out,
            src_v, dst_v, adj_v, rows_v, hid_sh, sem):
        cid = lax.axis_index("c")
        sid = lax.axis_index("s")
        wid = sid * _NC + cid
        r0 = sid * _ROWS_PER_TILE

        pltpu.sync_copy(zeros_h.at[pl.ds(r0, _ROWS_PER_TILE)],
                        hid_sh.at[pl.ds(r0, _ROWS_PER_TILE)])

        @pl.when(sid == 0)
        def _zero_tail():
            pltpu.sync_copy(zeros_h.at[pl.ds(_TAIL_BASE, _TAIL)],
                            hid_sh.at[pl.ds(_TAIL_BASE, _TAIL)])
        plsc.subcore_barrier()

        nw_chunks = base_chunks + jnp.where(wid < extra, 1, 0)

        def chunk_body(k, carry):
            base = (wid + k * _NW) * _C
            pltpu.sync_copy(src_h.at[pl.ds(base, _C)], src_v)
            pltpu.sync_copy(dst_h.at[pl.ds(base, _C)], dst_v)
            pltpu.sync_copy(adj_h.at[pl.ds(base, _C)], adj_v)
            pltpu.async_copy(support.at[src_v], rows_v, sem).wait()

            def grp_body(g, c2):
                a16 = adj_v[pl.ds(g * 16, 16)]
                for i in range(16):
                    bc = lax.gather(
                        a16, jnp.full((16, 1), i, jnp.int32),
                        lax.GatherDimensionNumbers(
                            offset_dims=(), collapsed_slice_dims=(0,),
                            start_index_map=(0,)),
                        (1,), mode=lax.GatherScatterMode.PROMISE_IN_BOUNDS)
                    r = g * 16 + i
                    for j in range(_D // 16):
                        sl = pl.ds(j * 16, 16)
                        rows_v[r, sl] = rows_v[r, sl] * bc
                return c2
            lax.fori_loop(0, _C // 16, grp_body, 0)

            pltpu.sync_copy(rows_v, hid_sh.at[dst_v], add=True)
            return carry
        lax.fori_loop(0, nw_chunks, chunk_body, 0)

        plsc.subcore_barrier()
        pltpu.sync_copy(hid_sh.at[pl.ds(r0, _ROWS_PER_TILE)],
                        out.at[cid, pl.ds(r0, _ROWS_PER_TILE)])

        @pl.when(sid == 0)
        def _out_tail():
            pltpu.sync_copy(hid_sh.at[pl.ds(_TAIL_BASE, _TAIL)],
                            out.at[cid, pl.ds(_TAIL_BASE, _TAIL)])
    return agg


def kernel(edge_index, adj_values, feats, W, b, ln_gamma, ln_beta):
    E = adj_values.shape[0]
    src = edge_index[1].astype(jnp.int32)
    dst = edge_index[0].astype(jnp.int32)

    support = pl.pallas_call(
        _linear_kernel,
        grid=(10,),
        in_specs=[
            pl.BlockSpec((1000, _D), lambda i: (i, 0)),
            pl.BlockSpec((_D, _D), lambda i: (0, 0)),
            pl.BlockSpec((1, _D), lambda i: (0, 0)),
        ],
        out_specs=pl.BlockSpec((1000, _D), lambda i: (i, 0)),
        out_shape=jax.ShapeDtypeStruct((_N, _D), jnp.float32),
    )(feats, W, b.reshape(1, _D))

    zeros = jnp.zeros((_N, _D), jnp.float32)
    partials = _make_agg(E)(support, src, dst, adj_values, zeros)

    out = pl.pallas_call(
        _ln_kernel,
        grid=(10,),
        in_specs=[
            pl.BlockSpec((_NC, 1000, _D), lambda i: (0, i, 0)),
            pl.BlockSpec((1, _D), lambda i: (0, 0)),
            pl.BlockSpec((1, _D), lambda i: (0, 0)),
        ],
        out_specs=pl.BlockSpec((1000, _D), lambda i: (i, 0)),
        out_shape=jax.ShapeDtypeStruct((_N, _D), jnp.float32),
    )(partials, ln_gamma.reshape(1, _D), ln_beta.reshape(1, _D))
    return out


# speedup vs baseline: 10.8836x; 2.0592x over previous
import functools

import jax
import jax.numpy as jnp
from jax import lax
from jax.experimental import pallas as pl
from jax.experimental.pallas import tpu as pltpu
from jax.experimental.pallas import tpu_sc as plsc

_N = 10000
_D = 128
_NC = 2
_NS = 16
_NW = _NC * _NS
_C = 128
_NB = 4
_NR = 2
_ROWS_PER_TILE = 624
_TAIL_BASE = _ROWS_PER_TILE * _NS
_TAIL = _N - _TAIL_BASE


def _linear_kernel(f_ref, w_ref, b_ref, o_ref):
    o_ref[...] = lax.dot_general(
        f_ref[...], w_ref[...],
        dimension_numbers=(((1,), (1,)), ((), ())),
        preferred_element_type=jnp.float32,
        precision=lax.Precision.HIGHEST,
    ) + b_ref[...]


def _ln_kernel(p_ref, g_ref, be_ref, o_ref):
    h = p_ref[0] + p_ref[1]
    m = jnp.mean(h, axis=-1, keepdims=True)
    c = h - m
    v = jnp.mean(c * c, axis=-1, keepdims=True)
    y = c * lax.rsqrt(v + 1e-5) * g_ref[...] + be_ref[...]
    o_ref[...] = jnp.maximum(y, 0.0)


def _make_agg(E):
    nchunks = E // _C
    base_chunks = nchunks // _NW
    extra = nchunks % _NW
    npad = base_chunks + (1 if extra else 0)
    npad += (-npad) % _NB
    mesh = plsc.VectorSubcoreMesh(core_axis_name="c", subcore_axis_name="s")

    @functools.partial(
        pl.kernel, mesh=mesh,
        out_type=jax.ShapeDtypeStruct((_NC, _N, _D), jnp.float32),
        scratch_types=[
            pltpu.VMEM((_NB, _C), jnp.int32),
            pltpu.VMEM((_NB, _C), jnp.int32),
            pltpu.VMEM((_NB, _C), jnp.float32),
            pltpu.VMEM((_NR, _C, _D), jnp.float32),
            pltpu.VMEM_SHARED((_N, _D), jnp.float32),
            pltpu.SemaphoreType.DMA((_NB,)),
            pltpu.SemaphoreType.DMA((_NR,)),
            pltpu.SemaphoreType.DMA((_NR,)),
        ],
    )
    def agg(support, src_h, dst_h, adj_h, zeros_h, out,
            src_v, dst_v, adj_v, rows_v, hid_sh, isem, gsem, ssem):
        cid = lax.axis_index("c")
        sid = lax.axis_index("s")
        wid = sid * _NC + cid
        r0 = sid * _ROWS_PER_TILE

        pltpu.sync_copy(zeros_h.at[pl.ds(r0, _ROWS_PER_TILE)],
                        hid_sh.at[pl.ds(r0, _ROWS_PER_TILE)])

        @pl.when(sid == 0)
        def _zero_tail():
            pltpu.sync_copy(zeros_h.at[pl.ds(_TAIL_BASE, _TAIL)],
                            hid_sh.at[pl.ds(_TAIL_BASE, _TAIL)])
        plsc.subcore_barrier()

        n_w = base_chunks + jnp.where(wid < extra, 1, 0)

        def start_idx(m, b):
            base = (wid + m * _NW) * _C
            pltpu.async_copy(src_h.at[pl.ds(base, _C)], src_v.at[b], isem.at[b])
            pltpu.async_copy(dst_h.at[pl.ds(base, _C)], dst_v.at[b], isem.at[b])
            pltpu.async_copy(adj_h.at[pl.ds(base, _C)], adj_v.at[b], isem.at[b])

        def wait_idx(b):
            pltpu.make_async_copy(src_h.at[pl.ds(0, _C)], src_v.at[b], isem.at[b]).wait()
            pltpu.make_async_copy(dst_h.at[pl.ds(0, _C)], dst_v.at[b], isem.at[b]).wait()
            pltpu.make_async_copy(adj_h.at[pl.ds(0, _C)], adj_v.at[b], isem.at[b]).wait()

        def start_gather(br, bi):
            pltpu.async_copy(support.at[src_v.at[bi]], rows_v.at[br], gsem.at[br])

        def wait_gather(br, bi):
            pltpu.make_async_copy(support.at[src_v.at[bi]], rows_v.at[br], gsem.at[br]).wait()

        def start_scatter(br, bi):
            pltpu.async_copy(rows_v.at[br], hid_sh.at[dst_v.at[bi]], ssem.at[br], add=True)

        def wait_scatter(br, bi):
            pltpu.make_async_copy(rows_v.at[br], hid_sh.at[dst_v.at[bi]], ssem.at[br]).wait()

        def scale(br, bi):
            def grp_body(g, c2):
                a16 = adj_v[bi, pl.ds(g * 16, 16)]
                for i in range(16):
                    bc = lax.gather(
                        a16, jnp.full((16, 1), i, jnp.int32),
                        lax.GatherDimensionNumbers(
                            offset_dims=(), collapsed_slice_dims=(0,),
                            start_index_map=(0,)),
                        (1,), mode=lax.GatherScatterMode.PROMISE_IN_BOUNDS)
                    r = g * 16 + i
                    for j in range(_D // 16):
                        sl = pl.ds(j * 16, 16)
                        rows_v[br, r, sl] = rows_v[br, r, sl] * bc
                return c2
            lax.fori_loop(0, _C // 16, grp_body, 0)

        start_idx(0, 0)
        start_idx(1, 1)
        wait_idx(0)
        start_gather(0, 0)

        def quad_body(k, carry):
            for h in range(_NB):
                m = k * _NB + h
                br, bi = h % _NR, h
                nbr, nbi = (h + 1) % _NR, (h + 1) % _NB
                pbr, pbi = (h - 1) % _NR, (h - 1) % _NB
                nnbi = (h + 2) % _NB

                @pl.when((m >= 1) & (m <= n_w))
                def _w_sc():
                    wait_scatter(pbr, pbi)

                @pl.when(m + 2 < n_w)
                def _s_idx():
                    start_idx(m + 2, nnbi)

                @pl.when(m + 1 < n_w)
                def _s_g():
                    wait_idx(nbi)
                    start_gather(nbr, nbi)

                @pl.when(m < n_w)
                def _proc():
                    wait_gather(br, bi)
                    scale(br, bi)
                    start_scatter(br, bi)
            return carry
        lax.fori_loop(0, npad // _NB, quad_body, 0)

        plsc.subcore_barrier()
        pltpu.sync_copy(hid_sh.at[pl.ds(r0, _ROWS_PER_TILE)],
                        out.at[cid, pl.ds(r0, _ROWS_PER_TILE)])

        @pl.when(sid == 0)
        def _out_tail():
            pltpu.sync_copy(hid_sh.at[pl.ds(_TAIL_BASE, _TAIL)],
                            out.at[cid, pl.ds(_TAIL_BASE, _TAIL)])
    return agg


def kernel(edge_index, adj_values, feats, W, b, ln_gamma, ln_beta):
    E = adj_values.shape[0]
    src = edge_index[1].astype(jnp.int32)
    dst = edge_index[0].astype(jnp.int32)

    support = pl.pallas_call(
        _linear_kernel,
        grid=(10,),
        in_specs=[
            pl.BlockSpec((1000, _D), lambda i: (i, 0)),
            pl.BlockSpec((_D, _D), lambda i: (0, 0)),
            pl.BlockSpec((1, _D), lambda i: (0, 0)),
        ],
        out_specs=pl.BlockSpec((1000, _D), lambda i: (i, 0)),
        out_shape=jax.ShapeDtypeStruct((_N, _D), jnp.float32),
    )(feats, W, b.reshape(1, _D))

    zeros = jnp.zeros((_N, _D), jnp.float32)
    partials = _make_agg(E)(support, src, dst, adj_values, zeros)

    out = pl.pallas_call(
        _ln_kernel,
        grid=(10,),
        in_specs=[
            pl.BlockSpec((_NC, 1000, _D), lambda i: (0, i, 0)),
            pl.BlockSpec((1, _D), lambda i: (0, 0)),
            pl.BlockSpec((1, _D), lambda i: (0, 0)),
        ],
        out_specs=pl.BlockSpec((1000, _D), lambda i: (i, 0)),
        out_shape=jax.ShapeDtypeStruct((_N, _D), jnp.float32),
    )(partials, ln_gamma.reshape(1, _D), ln_beta.reshape(1, _D))
    return out


# speedup vs baseline: 11.5046x; 1.0571x over previous
import functools

import jax
import jax.numpy as jnp
from jax import lax
from jax.experimental import pallas as pl
from jax.experimental.pallas import tpu as pltpu
from jax.experimental.pallas import tpu_sc as plsc

_N = 10000
_D = 128
_NC = 2
_NS = 16
_NW = _NC * _NS
_C = 128
_NB = 4
_NR = 3
_UNROLL = 12
_ROWS_PER_TILE = 624
_TAIL_BASE = _ROWS_PER_TILE * _NS
_TAIL = _N - _TAIL_BASE


def _linear_kernel(f_ref, w_ref, b_ref, o_ref):
    o_ref[...] = lax.dot_general(
        f_ref[...], w_ref[...],
        dimension_numbers=(((1,), (1,)), ((), ())),
        preferred_element_type=jnp.float32,
        precision=lax.Precision.HIGHEST,
    ) + b_ref[...]


def _ln_kernel(p_ref, g_ref, be_ref, o_ref):
    h = p_ref[0] + p_ref[1]
    m = jnp.mean(h, axis=-1, keepdims=True)
    c = h - m
    v = jnp.mean(c * c, axis=-1, keepdims=True)
    y = c * lax.rsqrt(v + 1e-5) * g_ref[...] + be_ref[...]
    o_ref[...] = jnp.maximum(y, 0.0)


def _make_agg(E):
    nchunks = E // _C
    base_chunks = nchunks // _NW
    extra = nchunks % _NW
    npad = base_chunks + (1 if extra else 0) + 2
    npad += (-npad) % _UNROLL
    mesh = plsc.VectorSubcoreMesh(core_axis_name="c", subcore_axis_name="s")

    @functools.partial(
        pl.kernel, mesh=mesh,
        out_type=jax.ShapeDtypeStruct((_NC, _N, _D), jnp.float32),
        scratch_types=[
            pltpu.VMEM((_NB, _C), jnp.int32),
            pltpu.VMEM((_NB, _C), jnp.int32),
            pltpu.VMEM((_NB, _C), jnp.float32),
            pltpu.VMEM((_NR, _C, _D), jnp.float32),
            pltpu.VMEM_SHARED((_N, _D), jnp.float32),
            pltpu.SemaphoreType.DMA((_NB,)),
            pltpu.SemaphoreType.DMA((_NR,)),
            pltpu.SemaphoreType.DMA((_NR,)),
        ],
    )
    def agg(support, src_h, dst_h, adj_h, zeros_h, out,
            src_v, dst_v, adj_v, rows_v, hid_sh, isem, gsem, ssem):
        cid = lax.axis_index("c")
        sid = lax.axis_index("s")
        wid = sid * _NC + cid
        r0 = sid * _ROWS_PER_TILE

        pltpu.sync_copy(zeros_h.at[pl.ds(r0, _ROWS_PER_TILE)],
                        hid_sh.at[pl.ds(r0, _ROWS_PER_TILE)])

        @pl.when(sid == 0)
        def _zero_tail():
            pltpu.sync_copy(zeros_h.at[pl.ds(_TAIL_BASE, _TAIL)],
                            hid_sh.at[pl.ds(_TAIL_BASE, _TAIL)])
        plsc.subcore_barrier()

        n_w = base_chunks + jnp.where(wid < extra, 1, 0)

        def start_idx(m, b):
            base = (wid + m * _NW) * _C
            pltpu.async_copy(src_h.at[pl.ds(base, _C)], src_v.at[b], isem.at[b])
            pltpu.async_copy(dst_h.at[pl.ds(base, _C)], dst_v.at[b], isem.at[b])
            pltpu.async_copy(adj_h.at[pl.ds(base, _C)], adj_v.at[b], isem.at[b])

        def wait_idx(b):
            pltpu.make_async_copy(src_h.at[pl.ds(0, _C)], src_v.at[b], isem.at[b]).wait()
            pltpu.make_async_copy(dst_h.at[pl.ds(0, _C)], dst_v.at[b], isem.at[b]).wait()
            pltpu.make_async_copy(adj_h.at[pl.ds(0, _C)], adj_v.at[b], isem.at[b]).wait()

        def start_gather(br, bi):
            pltpu.async_copy(support.at[src_v.at[bi]], rows_v.at[br], gsem.at[br])

        def wait_gather(br, bi):
            pltpu.make_async_copy(support.at[src_v.at[bi]], rows_v.at[br], gsem.at[br]).wait()

        def start_scatter(br, bi):
            pltpu.async_copy(rows_v.at[br], hid_sh.at[dst_v.at[bi]], ssem.at[br], add=True)

        def wait_scatter(br, bi):
            pltpu.make_async_copy(rows_v.at[br], hid_sh.at[dst_v.at[bi]], ssem.at[br]).wait()

        def scale(br, bi):
            def grp_body(g, c2):
                a16 = adj_v[bi, pl.ds(g * 16, 16)]
                for i in range(16):
                    bc = lax.gather(
                        a16, jnp.full((16, 1), i, jnp.int32),
                        lax.GatherDimensionNumbers(
                            offset_dims=(), collapsed_slice_dims=(0,),
                            start_index_map=(0,)),
                        (1,), mode=lax.GatherScatterMode.PROMISE_IN_BOUNDS)
                    r = g * 16 + i
                    for j in range(_D // 16):
                        sl = pl.ds(j * 16, 16)
                        rows_v[br, r, sl] = rows_v[br, r, sl] * bc
                return c2
            lax.fori_loop(0, _C // 16, grp_body, 0)

        start_idx(0, 0)
        start_idx(1, 1)
        wait_idx(0)
        start_gather(0, 0)

        def unrolled_body(k, carry):
            for h in range(_UNROLL):
                m = k * _UNROLL + h
                br, bi = h % _NR, h % _NB
                nbr, nbi = (h + 1) % _NR, (h + 1) % _NB
                pbr, pbi = (h - 2) % _NR, (h - 2) % _NB
                nnbi = (h + 2) % _NB

                @pl.when((m >= 2) & (m < n_w + 2))
                def _w_sc():
                    wait_scatter(pbr, pbi)

                @pl.when(m + 2 < n_w)
                def _s_idx():
                    start_idx(m + 2, nnbi)

                @pl.when(m + 1 < n_w)
                def _s_g():
                    wait_idx(nbi)
                    start_gather(nbr, nbi)

                @pl.when(m < n_w)
                def _proc():
                    wait_gather(br, bi)
                    scale(br, bi)
                    start_scatter(br, bi)
            return carry
        lax.fori_loop(0, npad // _UNROLL, unrolled_body, 0)

        plsc.subcore_barrier()
        pltpu.sync_copy(hid_sh.at[pl.ds(r0, _ROWS_PER_TILE)],
                        out.at[cid, pl.ds(r0, _ROWS_PER_TILE)])

        @pl.when(sid == 0)
        def _out_tail():
            pltpu.sync_copy(hid_sh.at[pl.ds(_TAIL_BASE, _TAIL)],
                            out.at[cid, pl.ds(_TAIL_BASE, _TAIL)])
    return agg


def kernel(edge_index, adj_values, feats, W, b, ln_gamma, ln_beta):
    E = adj_values.shape[0]
    src = edge_index[1].astype(jnp.int32)
    dst = edge_index[0].astype(jnp.int32)

    support = pl.pallas_call(
        _linear_kernel,
        grid=(10,),
        in_specs=[
            pl.BlockSpec((1000, _D), lambda i: (i, 0)),
            pl.BlockSpec((_D, _D), lambda i: (0, 0)),
            pl.BlockSpec((1, _D), lambda i: (0, 0)),
        ],
        out_specs=pl.BlockSpec((1000, _D), lambda i: (i, 0)),
        out_shape=jax.ShapeDtypeStruct((_N, _D), jnp.float32),
    )(feats, W, b.reshape(1, _D))

    zeros = jnp.zeros((_N, _D), jnp.float32)
    partials = _make_agg(E)(support, src, dst, adj_values, zeros)

    out = pl.pallas_call(
        _ln_kernel,
        grid=(10,),
        in_specs=[
            pl.BlockSpec((_NC, 1000, _D), lambda i: (0, i, 0)),
            pl.BlockSpec((1, _D), lambda i: (0, 0)),
            pl.BlockSpec((1, _D), lambda i: (0, 0)),
        ],
        out_specs=pl.BlockSpec((1000, _D), lambda i: (i, 0)),
        out_shape=jax.ShapeDtypeStruct((_N, _D), jnp.float32),
    )(partials, ln_gamma.reshape(1, _D), ln_beta.reshape(1, _D))
    return out
